# baseline (speedup 1.0000x reference)
import functools

import jax
import jax.numpy as jnp
from jax.experimental import pallas as pl
from jax.experimental.pallas import tpu as pltpu

_EPS = 1e-5
_SLOPE = 0.01


def _col2im(taps, bN, H, W, bC):
    t4 = taps.reshape(bN, H, W, 16 * bC)
    g = 4 * bC
    t0 = t4[..., 0 * g:1 * g]
    t1 = t4[..., 1 * g:2 * g]
    t2 = t4[..., 2 * g:3 * g]
    t3 = t4[..., 3 * g:4 * g]
    zrow = jnp.zeros((bN, 1, W, g), taps.dtype)
    r0 = t1 + jnp.concatenate([zrow, t3[:, :-1]], axis=1)
    r1 = jnp.concatenate([t0[:, 1:], zrow], axis=1) + t2
    x = jnp.stack([r0, r1], axis=2).reshape(bN, 2 * H, W, g)
    s0 = x[..., 0 * bC:1 * bC]
    s1 = x[..., 1 * bC:2 * bC]
    s2 = x[..., 2 * bC:3 * bC]
    s3 = x[..., 3 * bC:4 * bC]
    zcol = jnp.zeros((bN, 2 * H, 1, bC), taps.dtype)
    w0 = s1 + jnp.concatenate([zcol, s3[:, :, :-1]], axis=2)
    w1 = jnp.concatenate([s0[:, :, 1:], zcol], axis=2) + s2
    return w0, w1


def _bn_lrelu_input(x_ref, s_ref, q_ref, g_ref, b_ref, cnt):
    bN, H, W, Cin = x_ref.shape
    inv = 1.0 / cnt
    s_tot = jnp.sum(s_ref[...], axis=(0, 1)).reshape(1, Cin)
    q_tot = jnp.sum(q_ref[...], axis=(0, 1)).reshape(1, Cin)
    mean = s_tot * inv
    var = q_tot * inv - mean * mean
    scale = g_ref[...] * jax.lax.rsqrt(var + _EPS)
    shift = b_ref[...] - mean * scale
    a = x_ref[...].astype(jnp.float32).reshape(bN * H * W, Cin) * scale + shift
    return jnp.where(a > 0, a, _SLOPE * a).astype(jnp.bfloat16)


def _mid_kernel(x_ref, s_ref, q_ref, g_ref, b_ref, w_ref,
                y_ref, so_ref, qo_ref, *, cnt):
    bN, H, W, Cin = x_ref.shape
    bC = y_ref.shape[3]
    a = _bn_lrelu_input(x_ref, s_ref, q_ref, g_ref, b_ref, cnt)
    w2 = w_ref[...].reshape(w_ref.shape[0], 16 * bC)
    taps = jnp.dot(a, w2, preferred_element_type=jnp.float32)
    w0, w1 = _col2im(taps, bN, H, W, bC)
    y_ref[:, :, 0::2, :] = w0
    y_ref[:, :, 1::2, :] = w1
    so_ref[...] = (jnp.sum(w0, axis=(0, 1, 2)) +
                   jnp.sum(w1, axis=(0, 1, 2))).reshape(1, 1, bC)
    qo_ref[...] = (jnp.sum(w0 * w0, axis=(0, 1, 2)) +
                   jnp.sum(w1 * w1, axis=(0, 1, 2))).reshape(1, 1, bC)


def _last_kernel(x_ref, s_ref, q_ref, g_ref, b_ref, w_ref, y_ref, *, cnt):
    bN, H, W, Cin = x_ref.shape
    Cout = y_ref.shape[0]
    a = _bn_lrelu_input(x_ref, s_ref, q_ref, g_ref, b_ref, cnt)
    tapsT = jax.lax.dot_general(w_ref[...].astype(jnp.bfloat16), a,
                                (((0,), (1,)), ((), ())),
                                preferred_element_type=jnp.float32)
    t5 = tapsT.reshape(4, 4 * Cout, bN, H, W)
    zr = jnp.zeros((4 * Cout, bN, 1, W), jnp.float32)
    r0 = t5[1] + jnp.concatenate([zr, t5[3][:, :, :-1]], axis=2)
    r1 = jnp.concatenate([t5[0][:, :, 1:], zr], axis=2) + t5[2]
    x = jnp.stack([r0, r1], axis=3).reshape(4, Cout, bN, 2 * H, W)
    zc = jnp.zeros((Cout, bN, 2 * H, 1), jnp.float32)
    w0 = x[1] + jnp.concatenate([zc, x[3][:, :, :, :-1]], axis=3)
    w1 = jnp.concatenate([x[0][:, :, :, 1:], zc], axis=3) + x[2]
    y_ref[...] = jnp.tanh(jnp.stack([w0, w1], axis=2))


def _pick_bn(N, H, W, bC):
    for bN in (32, 16, 8, 4, 2, 1):
        if N % bN:
            continue
        if bN * H * W <= 8192 and bN * H * W * 16 * bC <= 4 * 1024 * 1024:
            return bN
    return 1


def _conv_layer(x, s, q, g, b, w, *, cnt, last=False):
    N, H, W, Cin = x.shape
    if last:
        Cout = w.shape[1] // 16
        bC, gj = Cout, 1
    else:
        Cout = w.shape[2]
        bC = 128 if (Cout % 128 == 0 and Cout > 128) else Cout
        gj = Cout // bC
    bN = _pick_bn(N, H, W, bC)
    gi = N // bN
    Ho, Wo = 2 * H, 2 * W

    in_specs = [
        pl.BlockSpec((bN, H, W, Cin), lambda i, j: (i, 0, 0, 0)),
        pl.BlockSpec(s.shape, lambda i, j: (0, 0, 0)),
        pl.BlockSpec(q.shape, lambda i, j: (0, 0, 0)),
        pl.BlockSpec((1, Cin), lambda i, j: (0, 0)),
        pl.BlockSpec((1, Cin), lambda i, j: (0, 0)),
    ]
    if last:
        in_specs.append(pl.BlockSpec(w.shape, lambda i, j: (0, 0)))
        body = functools.partial(_last_kernel, cnt=cnt)
        out_shape = jax.ShapeDtypeStruct((Cout, N, 2, Ho, W), jnp.float32)
        out_specs = pl.BlockSpec((Cout, bN, 2, Ho, W),
                                 lambda i, j: (0, i, 0, 0, 0))
    else:
        in_specs.append(pl.BlockSpec((Cin, 16, bC), lambda i, j: (0, 0, j)))
        body = functools.partial(_mid_kernel, cnt=cnt)
        out_shape = (
            jax.ShapeDtypeStruct((N, Ho, Wo, Cout), jnp.float32),
            jax.ShapeDtypeStruct((gi, 1, Cout), jnp.float32),
            jax.ShapeDtypeStruct((gi, 1, Cout), jnp.float32),
        )
        out_specs = (
            pl.BlockSpec((bN, Ho, Wo, bC), lambda i, j: (i, 0, 0, j)),
            pl.BlockSpec((1, 1, bC), lambda i, j: (i, 0, j)),
            pl.BlockSpec((1, 1, bC), lambda i, j: (i, 0, j)),
        )
    return pl.pallas_call(
        body,
        grid=(gi, gj),
        in_specs=in_specs,
        out_specs=out_specs,
        out_shape=out_shape,
        compiler_params=pltpu.CompilerParams(
            dimension_semantics=("parallel", "parallel"),
            vmem_limit_bytes=100 * 1024 * 1024),
    )(x, s, q, g, b, w)


def _l12_kernel(x_ref, w1_ref, g_ref, b_ref, w_ref,
                y_ref, so_ref, qo_ref, *, cnt):
    N = x_ref.shape[0]
    C1 = w1_ref.shape[1] // 16
    bC = y_ref.shape[3]
    taps1 = jnp.dot(x_ref[...], w1_ref[...],
                    preferred_element_type=jnp.float32)
    y1 = taps1.reshape(N * 16, C1)
    inv = 1.0 / cnt
    mean = jnp.sum(y1, axis=0, keepdims=True) * inv
    var = jnp.sum(y1 * y1, axis=0, keepdims=True) * inv - mean * mean
    scale = g_ref[...] * jax.lax.rsqrt(var + _EPS)
    shift = b_ref[...] - mean * scale
    a = y1 * scale + shift
    a = jnp.where(a > 0, a, _SLOPE * a).astype(jnp.bfloat16)
    w2 = w_ref[...].reshape(w_ref.shape[0], 16 * bC)
    taps = jnp.dot(a, w2, preferred_element_type=jnp.float32)
    w0, w1 = _col2im(taps, N, 4, 4, bC)
    y_ref[:, :, 0::2, :] = w0
    y_ref[:, :, 1::2, :] = w1
    so_ref[...] = (jnp.sum(w0, axis=(0, 1, 2)) +
                   jnp.sum(w1, axis=(0, 1, 2))).reshape(1, 1, bC)
    qo_ref[...] = (jnp.sum(w0 * w0, axis=(0, 1, 2)) +
                   jnp.sum(w1 * w1, axis=(0, 1, 2))).reshape(1, 1, bC)


def kernel(wm1, wm2, wm3, wm4, wm5, gamma1, beta1, gamma2, beta2,
           gamma3, beta3, gamma4, beta4, noise, condition):
    N = noise.shape[0]
    x = jnp.concatenate([noise.reshape(N, -1), condition.reshape(N, -1)],
                        axis=1).astype(jnp.bfloat16)
    C1 = wm1.shape[1] // 16

    def wprep(wm):
        cin, k16c = wm.shape
        cout = k16c // 16
        ck = 128 if cout % 128 == 0 and cout >= 128 else cout
        nj = cout // ck
        del nj
        return wm.astype(jnp.bfloat16).reshape(cin, 16, cout)

    w2p = wprep(wm2)
    C2 = w2p.shape[2]
    bC2 = 128 if (C2 % 128 == 0 and C2 > 128) else C2
    gj2 = C2 // bC2
    y2, s2, q2 = pl.pallas_call(
        functools.partial(_l12_kernel, cnt=float(N * 16)),
        grid=(1, gj2),
        in_specs=[
            pl.BlockSpec((N, x.shape[1]), lambda i, j: (0, 0)),
            pl.BlockSpec(wm1.shape, lambda i, j: (0, 0)),
            pl.BlockSpec((1, C1), lambda i, j: (0, 0)),
            pl.BlockSpec((1, C1), lambda i, j: (0, 0)),
            pl.BlockSpec((wm2.shape[0], 16, bC2), lambda i, j: (0, 0, j)),
        ],
        out_specs=(
            pl.BlockSpec((N, 8, 8, bC2), lambda i, j: (0, 0, 0, j)),
            pl.BlockSpec((1, 1, bC2), lambda i, j: (0, 0, j)),
            pl.BlockSpec((1, 1, bC2), lambda i, j: (0, 0, j)),
        ),
        out_shape=(
            jax.ShapeDtypeStruct((N, 8, 8, C2), jnp.float32),
            jax.ShapeDtypeStruct((1, 1, C2), jnp.float32),
            jax.ShapeDtypeStruct((1, 1, C2), jnp.float32),
        ),
        compiler_params=pltpu.CompilerParams(
            dimension_semantics=("parallel", "parallel"),
            vmem_limit_bytes=100 * 1024 * 1024),
    )(x, wm1.astype(jnp.bfloat16), gamma1.reshape(1, -1),
      beta1.reshape(1, -1), w2p)
    y3, s3, q3 = _conv_layer(y2, s2, q2, gamma2.reshape(1, -1),
                             beta2.reshape(1, -1), wprep(wm3),
                             cnt=float(N * 64))
    y4, s4, q4 = _conv_layer(y3, s3, q3, gamma3.reshape(1, -1),
                             beta3.reshape(1, -1), wprep(wm4),
                             cnt=float(N * 256))
    y5 = _conv_layer(y4, s4, q4, gamma4.reshape(1, -1),
                     beta4.reshape(1, -1), wm5, cnt=float(N * 1024),
                     last=True)
    C5, _, _, Ho, Vo = y5.shape
    return jnp.transpose(y5, (1, 0, 3, 4, 2)).reshape(N, C5, Ho, 2 * Vo)


# speedup vs baseline: 1.0934x; 1.0934x over previous
import functools

import jax
import jax.numpy as jnp
from jax.experimental import pallas as pl
from jax.experimental.pallas import tpu as pltpu

_EPS = 1e-5
_SLOPE = 0.01


def _col2im(taps, bN, H, W, bC):
    t4 = taps.reshape(bN, H, W, 16 * bC)
    g = 4 * bC
    t0 = t4[..., 0 * g:1 * g]
    t1 = t4[..., 1 * g:2 * g]
    t2 = t4[..., 2 * g:3 * g]
    t3 = t4[..., 3 * g:4 * g]
    zrow = jnp.zeros((bN, 1, W, g), taps.dtype)
    r0 = t1 + jnp.concatenate([zrow, t3[:, :-1]], axis=1)
    r1 = jnp.concatenate([t0[:, 1:], zrow], axis=1) + t2
    x = jnp.stack([r0, r1], axis=2).reshape(bN, 2 * H, W, g)
    s0 = x[..., 0 * bC:1 * bC]
    s1 = x[..., 1 * bC:2 * bC]
    s2 = x[..., 2 * bC:3 * bC]
    s3 = x[..., 3 * bC:4 * bC]
    zcol = jnp.zeros((bN, 2 * H, 1, bC), taps.dtype)
    w0 = s1 + jnp.concatenate([zcol, s3[:, :, :-1]], axis=2)
    w1 = jnp.concatenate([s0[:, :, 1:], zcol], axis=2) + s2
    return w0, w1


def _bn_lrelu_input(x_ref, s_ref, q_ref, g_ref, b_ref, cnt):
    bN, H, W, Cin = x_ref.shape
    inv = 1.0 / cnt
    s_tot = jnp.sum(s_ref[...], axis=(0, 1)).reshape(1, Cin)
    q_tot = jnp.sum(q_ref[...], axis=(0, 1)).reshape(1, Cin)
    mean = s_tot * inv
    var = q_tot * inv - mean * mean
    scale = g_ref[...] * jax.lax.rsqrt(var + _EPS)
    shift = b_ref[...] - mean * scale
    a = x_ref[...].astype(jnp.float32).reshape(bN * H * W, Cin) * scale + shift
    return jnp.where(a > 0, a, _SLOPE * a).astype(jnp.bfloat16)


def _mid_kernel(x_ref, s_ref, q_ref, g_ref, b_ref, w_ref,
                y_ref, so_ref, qo_ref, *, cnt):
    bN, H, W, Cin = x_ref.shape
    bC = y_ref.shape[3]
    a = _bn_lrelu_input(x_ref, s_ref, q_ref, g_ref, b_ref, cnt)
    w2 = w_ref[...].reshape(w_ref.shape[0], 16 * bC)
    taps = jnp.dot(a, w2, preferred_element_type=jnp.float32)
    w0, w1 = _col2im(taps, bN, H, W, bC)
    y_ref[:, :, 0::2, :] = w0
    y_ref[:, :, 1::2, :] = w1
    so_ref[...] = (jnp.sum(w0, axis=(0, 1, 2)) +
                   jnp.sum(w1, axis=(0, 1, 2))).reshape(1, 1, bC)
    qo_ref[...] = (jnp.sum(w0 * w0, axis=(0, 1, 2)) +
                   jnp.sum(w1 * w1, axis=(0, 1, 2))).reshape(1, 1, bC)


def _last_kernel(x_ref, s_ref, q_ref, g_ref, b_ref, w_ref, y_ref, *, cnt):
    bN, H, W, Cin = x_ref.shape
    Cout = y_ref.shape[0]
    a = _bn_lrelu_input(x_ref, s_ref, q_ref, g_ref, b_ref, cnt)
    tapsT = jax.lax.dot_general(w_ref[...].astype(jnp.bfloat16), a,
                                (((0,), (1,)), ((), ())),
                                preferred_element_type=jnp.float32)
    t5 = tapsT.reshape(4, 4 * Cout, bN, H, W)
    zr = jnp.zeros((4 * Cout, bN, 1, W), jnp.float32)
    r0 = t5[1] + jnp.concatenate([zr, t5[3][:, :, :-1]], axis=2)
    r1 = jnp.concatenate([t5[0][:, :, 1:], zr], axis=2) + t5[2]
    x = jnp.stack([r0, r1], axis=3).reshape(4, Cout, bN, 2 * H, W)
    zc = jnp.zeros((Cout, bN, 2 * H, 1), jnp.float32)
    w0 = x[1] + jnp.concatenate([zc, x[3][:, :, :, :-1]], axis=3)
    w1 = jnp.concatenate([x[0][:, :, :, 1:], zc], axis=3) + x[2]
    y_ref[...] = jnp.tanh(jnp.stack([w0, w1], axis=2))


def _pick_bn(N, H, W, bC):
    for bN in (32, 16, 8, 4, 2, 1):
        if N % bN:
            continue
        if bN * H * W <= 8192 and bN * H * W * 16 * bC <= 4 * 1024 * 1024:
            return bN
    return 1


def _conv_layer(x, s, q, g, b, w, *, cnt, last=False):
    N, H, W, Cin = x.shape
    if last:
        Cout = w.shape[1] // 16
        bC, gj = Cout, 1
    else:
        Cout = w.shape[2]
        bC = 128 if (Cout % 128 == 0 and Cout > 128) else Cout
        gj = Cout // bC
    bN = _pick_bn(N, H, W, bC)
    gi = N // bN
    Ho, Wo = 2 * H, 2 * W

    in_specs = [
        pl.BlockSpec((bN, H, W, Cin), lambda i, j: (i, 0, 0, 0)),
        pl.BlockSpec(s.shape, lambda i, j: (0, 0, 0)),
        pl.BlockSpec(q.shape, lambda i, j: (0, 0, 0)),
        pl.BlockSpec((1, Cin), lambda i, j: (0, 0)),
        pl.BlockSpec((1, Cin), lambda i, j: (0, 0)),
    ]
    if last:
        in_specs.append(pl.BlockSpec(w.shape, lambda i, j: (0, 0)))
        body = functools.partial(_last_kernel, cnt=cnt)
        out_shape = jax.ShapeDtypeStruct((Cout, N, 2, Ho, W), jnp.float32)
        out_specs = pl.BlockSpec((Cout, bN, 2, Ho, W),
                                 lambda i, j: (0, i, 0, 0, 0))
    else:
        in_specs.append(pl.BlockSpec((Cin, 16, bC), lambda i, j: (0, 0, j)))
        body = functools.partial(_mid_kernel, cnt=cnt)
        out_shape = (
            jax.ShapeDtypeStruct((N, Ho, Wo, Cout), jnp.float32),
            jax.ShapeDtypeStruct((gi, 1, Cout), jnp.float32),
            jax.ShapeDtypeStruct((gi, 1, Cout), jnp.float32),
        )
        out_specs = (
            pl.BlockSpec((bN, Ho, Wo, bC), lambda i, j: (i, 0, 0, j)),
            pl.BlockSpec((1, 1, bC), lambda i, j: (i, 0, j)),
            pl.BlockSpec((1, 1, bC), lambda i, j: (i, 0, j)),
        )
    return pl.pallas_call(
        body,
        grid=(gi, gj),
        in_specs=in_specs,
        out_specs=out_specs,
        out_shape=out_shape,
        compiler_params=pltpu.CompilerParams(
            dimension_semantics=("parallel", "parallel"),
            vmem_limit_bytes=100 * 1024 * 1024),
    )(x, s, q, g, b, w)


def _l1_kernel(x_ref, w_ref, y_ref, s_ref, q_ref):
    N, Cin = x_ref.shape
    C1 = w_ref.shape[1] // 16
    taps = jnp.dot(x_ref[...], w_ref[...].astype(jnp.bfloat16),
                   preferred_element_type=jnp.float32)
    y_ref[...] = taps.reshape(N, 4, 4, C1).astype(jnp.bfloat16)
    t3 = taps.reshape(N, 16, C1)
    s_ref[...] = jnp.sum(t3, axis=(0, 1)).reshape(1, 1, C1)
    q_ref[...] = jnp.sum(t3 * t3, axis=(0, 1)).reshape(1, 1, C1)


def kernel(wm1, wm2, wm3, wm4, wm5, gamma1, beta1, gamma2, beta2,
           gamma3, beta3, gamma4, beta4, noise, condition):
    N = noise.shape[0]
    x = jnp.concatenate([noise.reshape(N, -1), condition.reshape(N, -1)],
                        axis=1).astype(jnp.bfloat16)
    C1 = wm1.shape[1] // 16

    y1, s1, q1 = pl.pallas_call(
        _l1_kernel,
        out_shape=(jax.ShapeDtypeStruct((N, 4, 4, C1), jnp.bfloat16),
                   jax.ShapeDtypeStruct((1, 1, C1), jnp.float32),
                   jax.ShapeDtypeStruct((1, 1, C1), jnp.float32)),
    )(x, wm1)

    def wprep(wm):
        cin, k16c = wm.shape
        cout = k16c // 16
        ck = 128 if cout % 128 == 0 and cout >= 128 else cout
        nj = cout // ck
        del nj
        return wm.astype(jnp.bfloat16).reshape(cin, 16, cout)

    y2, s2, q2 = _conv_layer(y1, s1, q1, gamma1.reshape(1, -1),
                             beta1.reshape(1, -1), wprep(wm2),
                             cnt=float(N * 16))
    y3, s3, q3 = _conv_layer(y2, s2, q2, gamma2.reshape(1, -1),
                             beta2.reshape(1, -1), wprep(wm3),
                             cnt=float(N * 64))
    y4, s4, q4 = _conv_layer(y3, s3, q3, gamma3.reshape(1, -1),
                             beta3.reshape(1, -1), wprep(wm4),
                             cnt=float(N * 256))
    y5 = _conv_layer(y4, s4, q4, gamma4.reshape(1, -1),
                     beta4.reshape(1, -1), wm5, cnt=float(N * 1024),
                     last=True)
    C5, _, _, Ho, Vo = y5.shape
    return jnp.transpose(y5, (1, 0, 3, 4, 2)).reshape(N, C5, Ho, 2 * Vo)
